# Initial kernel scaffold; baseline (speedup 1.0000x reference)
import functools

import jax
import jax.numpy as jnp
from jax import lax
from jax.experimental import pallas as pl
from jax.experimental.pallas import tpu as pltpu
from jax.experimental.pallas import tpu_sc as plsc

N_NODES = 10000
N_EDGES = 320000
D_FEAT = 128
DIM = 32
N_GRAPHS = 64

NC = 2
NS = 16
NW = NC * NS
EPW = N_EDGES // NW
CH = 80
NCH = EPW // CH
RPS = N_NODES // NS


def _mm_body(x_ref, wr_ref, wo_ref, y_ref, r_ref):
    x = x_ref[...]
    dn = (((1,), (1,)), ((), ()))
    y_ref[...] = lax.dot_general(x, wr_ref[...], dn,
                                 preferred_element_type=jnp.float32)
    r_ref[...] = lax.dot_general(x, wo_ref[...], dn,
                                 preferred_element_type=jnp.float32)


def _sc_agg_body(y_hbm, src_hbm, dst_hbm, w_hbm, zeros_hbm, out_hbm,
                 src_v, dst_v, w_v, rows_v, agg_sh, sem):
    c = lax.axis_index("c")
    s = lax.axis_index("s")
    wid = s * NC + c

    pltpu.sync_copy(zeros_hbm.at[pl.ds(s * RPS, RPS)],
                    agg_sh.at[pl.ds(s * RPS, RPS)])
    plsc.subcore_barrier()

    pltpu.sync_copy(src_hbm.at[wid], src_v)
    pltpu.sync_copy(dst_hbm.at[wid], dst_v)
    pltpu.sync_copy(w_hbm.at[wid], w_v)

    def chunk_body(j, _):
        pltpu.async_copy(y_hbm.at[src_v.at[j]], rows_v, sem).wait()

        def edge_body(i, _):
            ws = w_v[j, i]
            rows_v[i, pl.ds(0, 16)] = rows_v[i, pl.ds(0, 16)] * ws
            rows_v[i, pl.ds(16, 16)] = rows_v[i, pl.ds(16, 16)] * ws
            return 0

        lax.fori_loop(0, CH, edge_body, 0)

        pltpu.sync_copy(rows_v, agg_sh.at[dst_v.at[j]], add=True)
        return 0

    lax.fori_loop(0, NCH, chunk_body, 0)
    plsc.subcore_barrier()

    pltpu.sync_copy(agg_sh.at[pl.ds(s * RPS, RPS)],
                    out_hbm.at[c, pl.ds(s * RPS, RPS)])


def _final_body(agg_ref, x_ref, wo_ref, brel_ref, batch_ref,
                w1_ref, b1_ref, w2_ref, b2_ref, out_ref):
    dn = (((1,), (1,)), ((), ()))
    agg = agg_ref[0] + agg_ref[1]
    r = lax.dot_general(x_ref[...], wo_ref[...], dn,
                        preferred_element_type=jnp.float32)
    h = jnp.maximum(agg + brel_ref[...] + r, 0.0)

    bt = batch_ref[...].reshape(1, N_NODES)
    gids = lax.broadcasted_iota(jnp.int32, (N_GRAPHS, N_NODES), 0)
    seg = (bt == gids).astype(jnp.float32)
    pooled = lax.dot_general(seg, h, (((1,), (0,)), ((), ())),
                             preferred_element_type=jnp.float32)

    z = jnp.maximum(lax.dot_general(pooled, w1_ref[...], dn,
                                    preferred_element_type=jnp.float32)
                    + b1_ref[...], 0.0)
    o = lax.dot_general(z, w2_ref[...], dn,
                        preferred_element_type=jnp.float32) + b2_ref[...]
    m = jnp.max(o, axis=1, keepdims=True)
    lo = o - m
    out_ref[...] = lo - jnp.log(jnp.sum(jnp.exp(lo), axis=1, keepdims=True))


def kernel(x, edge_index, batch, edge_weight, W_rel, b_rel, W_root,
           W1, b1, W2, b2):
    src = edge_index[0].astype(jnp.int32).reshape(NW, NCH, CH)
    dst = edge_index[1].astype(jnp.int32).reshape(NW, NCH, CH)
    ew = edge_weight.astype(jnp.float32).reshape(NW, NCH, CH)
    batch32 = batch.astype(jnp.int32).reshape(8, N_NODES // 8)
    zeros = jnp.zeros((N_NODES, DIM), jnp.float32)

    y, r = pl.pallas_call(
        _mm_body,
        out_shape=[jax.ShapeDtypeStruct((N_NODES, DIM), jnp.float32),
                   jax.ShapeDtypeStruct((N_NODES, DIM), jnp.float32)],
    )(x, W_rel, W_root)
    del r

    mesh = plsc.VectorSubcoreMesh(core_axis_name="c", subcore_axis_name="s")
    agg_parts = pl.kernel(
        _sc_agg_body,
        out_type=jax.ShapeDtypeStruct((NC, N_NODES, DIM), jnp.float32),
        mesh=mesh,
        scratch_types=[
            pltpu.VMEM((NCH, CH), jnp.int32),
            pltpu.VMEM((NCH, CH), jnp.int32),
            pltpu.VMEM((NCH, CH), jnp.float32),
            pltpu.VMEM((CH, DIM), jnp.float32),
            pltpu.VMEM_SHARED((N_NODES, DIM), jnp.float32),
            pltpu.SemaphoreType.DMA,
        ],
    )(y, src, dst, ew, zeros)

    out = pl.pallas_call(
        _final_body,
        out_shape=jax.ShapeDtypeStruct((N_GRAPHS, 2), jnp.float32),
    )(agg_parts, x, W_root, b_rel.reshape(1, DIM), batch32,
      W1, b1.reshape(1, DIM), W2, b2.reshape(1, 2))
    return out


# speedup vs baseline: 6.8810x; 6.8810x over previous
import functools

import jax
import jax.numpy as jnp
from jax import lax
from jax.experimental import pallas as pl
from jax.experimental.pallas import tpu as pltpu
from jax.experimental.pallas import tpu_sc as plsc

N_NODES = 10000
N_EDGES = 320000
D_FEAT = 128
DIM = 32
N_GRAPHS = 64

NC = 2
NS = 16
NW = NC * NS
CH = 80
NCH = 128
EPAD = NW * NCH * CH
NP = 10240
RPS = NP // NS


def _mm_body(x_ref, wr_ref, y_ref):
    dn = (((1,), (1,)), ((), ()))
    y_ref[...] = lax.dot_general(x_ref[...], wr_ref[...], dn,
                                 preferred_element_type=jnp.float32)


def _sc_agg_body(y_hbm, src_hbm, dst_hbm, w_hbm, zeros_hbm, out_hbm,
                 src_v, dst_v, w_v, rows_v, agg_sh, sem):
    c = lax.axis_index("c")
    s = lax.axis_index("s")
    wid = s * NC + c

    pltpu.sync_copy(zeros_hbm.at[pl.ds(s * RPS, RPS)],
                    agg_sh.at[pl.ds(s * RPS, RPS)])
    plsc.subcore_barrier()

    pltpu.sync_copy(src_hbm.at[wid], src_v)
    pltpu.sync_copy(dst_hbm.at[wid], dst_v)
    pltpu.sync_copy(w_hbm.at[wid], w_v)

    def chunk_body(j, _):
        pltpu.async_copy(y_hbm.at[src_v.at[j]], rows_v, sem).wait()

        def group_body(g, _):
            base = g * 16
            wv = w_v[j, pl.ds(base, 16)]
            for k in range(16):
                ws = wv[k]
                rows_v[base + k, pl.ds(0, 16)] = (
                    rows_v[base + k, pl.ds(0, 16)] * ws)
                rows_v[base + k, pl.ds(16, 16)] = (
                    rows_v[base + k, pl.ds(16, 16)] * ws)
            return 0

        lax.fori_loop(0, CH // 16, group_body, 0)

        pltpu.sync_copy(rows_v, agg_sh.at[dst_v.at[j]], add=True)
        return 0

    lax.fori_loop(0, NCH, chunk_body, 0)
    plsc.subcore_barrier()

    pltpu.sync_copy(agg_sh.at[pl.ds(s * RPS, RPS)],
                    out_hbm.at[c, pl.ds(s * RPS, RPS)])


def _final_body(agg_ref, x_ref, wo_ref, brel_ref, batch_ref,
                w1_ref, b1_ref, w2_ref, b2_ref, out_ref):
    dn = (((1,), (1,)), ((), ()))
    agg = agg_ref[0, :N_NODES, :] + agg_ref[1, :N_NODES, :]
    r = lax.dot_general(x_ref[...], wo_ref[...], dn,
                        preferred_element_type=jnp.float32)
    h = jnp.maximum(agg + brel_ref[...] + r, 0.0)

    bt = batch_ref[...]
    gids = lax.broadcasted_iota(jnp.int32, (N_GRAPHS, N_NODES), 0)
    seg = (bt == gids).astype(jnp.float32)
    pooled = lax.dot_general(seg, h, (((1,), (0,)), ((), ())),
                             preferred_element_type=jnp.float32)

    z = jnp.maximum(lax.dot_general(pooled, w1_ref[...], dn,
                                    preferred_element_type=jnp.float32)
                    + b1_ref[...], 0.0)
    o = lax.dot_general(z, w2_ref[...], dn,
                        preferred_element_type=jnp.float32) + b2_ref[...]
    m = jnp.max(o, axis=1, keepdims=True)
    lo = o - m
    out_ref[...] = lo - jnp.log(jnp.sum(jnp.exp(lo), axis=1, keepdims=True))


def kernel(x, edge_index, batch, edge_weight, W_rel, b_rel, W_root,
           W1, b1, W2, b2):
    pad = EPAD - N_EDGES
    ipad = jnp.zeros((pad,), jnp.int32)
    src = jnp.concatenate([edge_index[0].astype(jnp.int32), ipad])
    dst = jnp.concatenate([edge_index[1].astype(jnp.int32), ipad])
    ew = jnp.concatenate([edge_weight.astype(jnp.float32),
                          jnp.zeros((pad,), jnp.float32)])
    src = src.reshape(NW, NCH, CH)
    dst = dst.reshape(NW, NCH, CH)
    ew = ew.reshape(NW, NCH, CH)
    batch32 = batch.astype(jnp.int32).reshape(1, N_NODES)
    zeros = jnp.zeros((NP, DIM), jnp.float32)

    y = pl.pallas_call(
        _mm_body,
        out_shape=jax.ShapeDtypeStruct((N_NODES, DIM), jnp.float32),
    )(x, W_rel)

    mesh = plsc.VectorSubcoreMesh(core_axis_name="c", subcore_axis_name="s",
                                  num_cores=NC, num_subcores=NS)
    agg_parts = pl.kernel(
        _sc_agg_body,
        out_type=jax.ShapeDtypeStruct((NC, NP, DIM), jnp.float32),
        mesh=mesh,
        scratch_types=[
            pltpu.VMEM((NCH, CH), jnp.int32),
            pltpu.VMEM((NCH, CH), jnp.int32),
            pltpu.VMEM((NCH, CH), jnp.float32),
            pltpu.VMEM((CH, DIM), jnp.float32),
            pltpu.VMEM_SHARED((NP, DIM), jnp.float32),
            pltpu.SemaphoreType.DMA,
        ],
        compiler_params=pltpu.CompilerParams(use_tc_tiling_on_sc=False),
    )(y, src, dst, ew, zeros)

    out = pl.pallas_call(
        _final_body,
        out_shape=jax.ShapeDtypeStruct((N_GRAPHS, 2), jnp.float32),
    )(agg_parts, x, W_root, b_rel.reshape(1, DIM), batch32,
      W1, b1.reshape(1, DIM), W2, b2.reshape(1, 2))
    return out


# speedup vs baseline: 9.7692x; 1.4197x over previous
import functools

import jax
import jax.numpy as jnp
from jax import lax
from jax.experimental import pallas as pl
from jax.experimental.pallas import tpu as pltpu
from jax.experimental.pallas import tpu_sc as plsc

N_NODES = 10000
N_EDGES = 320000
D_FEAT = 128
DIM = 32
N_GRAPHS = 64

NC = 2
NS = 16
NW = NC * NS
CH = 128
NCH = 80
EPAD = NW * NCH * CH
NP = 10240
RPS = NP // NS


def _mm_body(x_ref, wr_ref, y_ref):
    dn = (((1,), (1,)), ((), ()))
    y_ref[...] = lax.dot_general(x_ref[...], wr_ref[...], dn,
                                 preferred_element_type=jnp.float32)


def _sc_agg_body(y_hbm, src_hbm, dst_hbm, w_hbm, zeros_hbm, out_hbm,
                 src_v, dst_v, w_v, rows0_v, rows1_v, agg_sh, sem0, sem1):
    c = lax.axis_index("c")
    s = lax.axis_index("s")
    wid = s * NC + c

    pltpu.sync_copy(zeros_hbm.at[pl.ds(s * RPS, RPS)],
                    agg_sh.at[pl.ds(s * RPS, RPS)])
    plsc.subcore_barrier()

    pltpu.sync_copy(src_hbm.at[wid], src_v)
    pltpu.sync_copy(dst_hbm.at[wid], dst_v)
    pltpu.sync_copy(w_hbm.at[wid], w_v)

    def scale(rows_v, j):
        def group_body(g, _):
            base = g * 16
            wv = w_v[j, pl.ds(base, 16)]
            for k in range(16):
                ws = wv[k]
                rows_v[base + k, pl.ds(0, 16)] = (
                    rows_v[base + k, pl.ds(0, 16)] * ws)
                rows_v[base + k, pl.ds(16, 16)] = (
                    rows_v[base + k, pl.ds(16, 16)] * ws)
            return 0

        lax.fori_loop(0, CH // 16, group_body, 0)

    def wait(rows_v, sem):
        pltpu.make_async_copy(y_hbm.at[pl.ds(0, CH)], rows_v, sem).wait()

    pltpu.async_copy(y_hbm.at[src_v.at[0]], rows0_v, sem0)

    def pair_body(p, _):
        j0 = 2 * p
        j1 = 2 * p + 1
        pltpu.async_copy(y_hbm.at[src_v.at[j1]], rows1_v, sem1)
        wait(rows0_v, sem0)
        scale(rows0_v, j0)
        pltpu.sync_copy(rows0_v, agg_sh.at[dst_v.at[j0]], add=True)
        jn = jnp.minimum(j0 + 2, NCH - 1)
        pltpu.async_copy(y_hbm.at[src_v.at[jn]], rows0_v, sem0)
        wait(rows1_v, sem1)
        scale(rows1_v, j1)
        pltpu.sync_copy(rows1_v, agg_sh.at[dst_v.at[j1]], add=True)
        return 0

    lax.fori_loop(0, NCH // 2, pair_body, 0)
    wait(rows0_v, sem0)
    plsc.subcore_barrier()

    pltpu.sync_copy(agg_sh.at[pl.ds(s * RPS, RPS)],
                    out_hbm.at[c, pl.ds(s * RPS, RPS)])


def _final_body(agg_ref, x_ref, wo_ref, brel_ref, batch_ref,
                w1_ref, b1_ref, w2_ref, b2_ref, out_ref):
    dn = (((1,), (1,)), ((), ()))
    agg = agg_ref[0, :N_NODES, :] + agg_ref[1, :N_NODES, :]
    r = lax.dot_general(x_ref[...], wo_ref[...], dn,
                        preferred_element_type=jnp.float32)
    h = jnp.maximum(agg + brel_ref[...] + r, 0.0)

    bt = batch_ref[...]
    gids = lax.broadcasted_iota(jnp.int32, (N_GRAPHS, N_NODES), 0)
    seg = (bt == gids).astype(jnp.float32)
    pooled = lax.dot_general(seg, h, (((1,), (0,)), ((), ())),
                             preferred_element_type=jnp.float32)

    z = jnp.maximum(lax.dot_general(pooled, w1_ref[...], dn,
                                    preferred_element_type=jnp.float32)
                    + b1_ref[...], 0.0)
    o = lax.dot_general(z, w2_ref[...], dn,
                        preferred_element_type=jnp.float32) + b2_ref[...]
    m = jnp.max(o, axis=1, keepdims=True)
    lo = o - m
    out_ref[...] = lo - jnp.log(jnp.sum(jnp.exp(lo), axis=1, keepdims=True))


def kernel(x, edge_index, batch, edge_weight, W_rel, b_rel, W_root,
           W1, b1, W2, b2):
    pad = EPAD - N_EDGES
    ipad = jnp.zeros((pad,), jnp.int32)
    src = jnp.concatenate([edge_index[0].astype(jnp.int32), ipad])
    dst = jnp.concatenate([edge_index[1].astype(jnp.int32), ipad])
    ew = jnp.concatenate([edge_weight.astype(jnp.float32),
                          jnp.zeros((pad,), jnp.float32)])
    src = src.reshape(NW, NCH, CH)
    dst = dst.reshape(NW, NCH, CH)
    ew = ew.reshape(NW, NCH, CH)
    batch32 = batch.astype(jnp.int32).reshape(1, N_NODES)
    zeros = jnp.zeros((NP, DIM), jnp.float32)

    y = pl.pallas_call(
        _mm_body,
        out_shape=jax.ShapeDtypeStruct((N_NODES, DIM), jnp.float32),
    )(x, W_rel)

    mesh = plsc.VectorSubcoreMesh(core_axis_name="c", subcore_axis_name="s",
                                  num_cores=NC, num_subcores=NS)
    agg_parts = pl.kernel(
        _sc_agg_body,
        out_type=jax.ShapeDtypeStruct((NC, NP, DIM), jnp.float32),
        mesh=mesh,
        scratch_types=[
            pltpu.VMEM((NCH, CH), jnp.int32),
            pltpu.VMEM((NCH, CH), jnp.int32),
            pltpu.VMEM((NCH, CH), jnp.float32),
            pltpu.VMEM((CH, DIM), jnp.float32),
            pltpu.VMEM((CH, DIM), jnp.float32),
            pltpu.VMEM_SHARED((NP, DIM), jnp.float32),
            pltpu.SemaphoreType.DMA,
            pltpu.SemaphoreType.DMA,
        ],
        compiler_params=pltpu.CompilerParams(use_tc_tiling_on_sc=False),
    )(y, src, dst, ew, zeros)

    out = pl.pallas_call(
        _final_body,
        out_shape=jax.ShapeDtypeStruct((N_GRAPHS, 2), jnp.float32),
    )(agg_parts, x, W_root, b_rel.reshape(1, DIM), batch32,
      W1, b1.reshape(1, DIM), W2, b2.reshape(1, 2))
    return out


# speedup vs baseline: 10.2578x; 1.0500x over previous
import functools

import jax
import jax.numpy as jnp
from jax import lax
from jax.experimental import pallas as pl
from jax.experimental.pallas import tpu as pltpu
from jax.experimental.pallas import tpu_sc as plsc

N_NODES = 10000
N_EDGES = 320000
D_FEAT = 128
DIM = 32
N_GRAPHS = 64

NC = 2
NS = 16
NW = NC * NS
CH = 128
NCH = 80
EPAD = NW * NCH * CH
NP = 10240
RPS = NP // NS


def _mm_body(x_ref, wr_ref, y_ref):
    dn = (((1,), (1,)), ((), ()))
    y = lax.dot_general(x_ref[...], wr_ref[...], dn,
                        preferred_element_type=jnp.float32)
    y_ref[0] = y
    y_ref[1] = y


def _sc_agg_body(y_hbm, src_hbm, dst_hbm, w_hbm, zeros_hbm, out_hbm,
                 src_v, dst_v, w_v, rows0_v, rows1_v, agg_sh, sem0, sem1):
    c = lax.axis_index("c")
    s = lax.axis_index("s")
    wid = c * NS + s
    y_c = y_hbm.at[c]

    pltpu.sync_copy(zeros_hbm.at[c, pl.ds(s * RPS, RPS)],
                    agg_sh.at[pl.ds(s * RPS, RPS)])
    plsc.subcore_barrier()

    pltpu.sync_copy(src_hbm.at[wid], src_v)
    pltpu.sync_copy(dst_hbm.at[wid], dst_v)
    pltpu.sync_copy(w_hbm.at[wid], w_v)

    def scale(rows_v, j):
        def group_body(g, _):
            base = g * 16
            wv = w_v[j, pl.ds(base, 16)]
            for k in range(16):
                ws = wv[k]
                rows_v[base + k, pl.ds(0, 16)] = (
                    rows_v[base + k, pl.ds(0, 16)] * ws)
                rows_v[base + k, pl.ds(16, 16)] = (
                    rows_v[base + k, pl.ds(16, 16)] * ws)
            return 0

        lax.fori_loop(0, CH // 16, group_body, 0)

    def wait(rows_v, sem):
        pltpu.make_async_copy(y_hbm.at[c, pl.ds(0, CH)], rows_v, sem).wait()

    pltpu.async_copy(y_c.at[src_v.at[0]], rows0_v, sem0)

    def pair_body(p, _):
        j0 = 2 * p
        j1 = 2 * p + 1
        pltpu.async_copy(y_c.at[src_v.at[j1]], rows1_v, sem1)
        wait(rows0_v, sem0)
        scale(rows0_v, j0)
        pltpu.sync_copy(rows0_v, agg_sh.at[dst_v.at[j0]], add=True)
        jn = jnp.minimum(j0 + 2, NCH - 1)
        pltpu.async_copy(y_c.at[src_v.at[jn]], rows0_v, sem0)
        wait(rows1_v, sem1)
        scale(rows1_v, j1)
        pltpu.sync_copy(rows1_v, agg_sh.at[dst_v.at[j1]], add=True)
        return 0

    lax.fori_loop(0, NCH // 2, pair_body, 0)
    wait(rows0_v, sem0)
    plsc.subcore_barrier()

    pltpu.sync_copy(agg_sh.at[pl.ds(s * RPS, RPS)],
                    out_hbm.at[c, pl.ds(s * RPS, RPS)])


def _final_body(agg_ref, x_ref, wo_ref, brel_ref, batch_ref,
                w1_ref, b1_ref, w2_ref, b2_ref, out_ref):
    dn = (((1,), (1,)), ((), ()))
    agg = agg_ref[0, :N_NODES, :] + agg_ref[1, :N_NODES, :]
    r = lax.dot_general(x_ref[...], wo_ref[...], dn,
                        preferred_element_type=jnp.float32)
    h = jnp.maximum(agg + brel_ref[...] + r, 0.0)

    bt = batch_ref[...]
    gids = lax.broadcasted_iota(jnp.int32, (N_GRAPHS, N_NODES), 0)
    seg = (bt == gids).astype(jnp.float32)
    pooled = lax.dot_general(seg, h, (((1,), (0,)), ((), ())),
                             preferred_element_type=jnp.float32)

    z = jnp.maximum(lax.dot_general(pooled, w1_ref[...], dn,
                                    preferred_element_type=jnp.float32)
                    + b1_ref[...], 0.0)
    o = lax.dot_general(z, w2_ref[...], dn,
                        preferred_element_type=jnp.float32) + b2_ref[...]
    m = jnp.max(o, axis=1, keepdims=True)
    lo = o - m
    out_ref[...] = lo - jnp.log(jnp.sum(jnp.exp(lo), axis=1, keepdims=True))


def kernel(x, edge_index, batch, edge_weight, W_rel, b_rel, W_root,
           W1, b1, W2, b2):
    pad = EPAD - N_EDGES
    ipad = jnp.zeros((pad,), jnp.int32)
    src = jnp.concatenate([edge_index[0].astype(jnp.int32), ipad])
    dst = jnp.concatenate([edge_index[1].astype(jnp.int32), ipad])
    ew = jnp.concatenate([edge_weight.astype(jnp.float32),
                          jnp.zeros((pad,), jnp.float32)])
    src = src.reshape(NW, NCH, CH)
    dst = dst.reshape(NW, NCH, CH)
    ew = ew.reshape(NW, NCH, CH)
    batch32 = batch.astype(jnp.int32).reshape(1, N_NODES)
    zeros = jnp.zeros((NC, NP, DIM), jnp.float32)

    y = pl.pallas_call(
        _mm_body,
        out_shape=jax.ShapeDtypeStruct((NC, N_NODES, DIM), jnp.float32),
    )(x, W_rel)

    mesh = plsc.VectorSubcoreMesh(core_axis_name="c", subcore_axis_name="s",
                                  num_cores=NC, num_subcores=NS)
    agg_parts = pl.kernel(
        _sc_agg_body,
        out_type=jax.ShapeDtypeStruct((NC, NP, DIM), jnp.float32),
        mesh=mesh,
        scratch_types=[
            pltpu.VMEM((NCH, CH), jnp.int32),
            pltpu.VMEM((NCH, CH), jnp.int32),
            pltpu.VMEM((NCH, CH), jnp.float32),
            pltpu.VMEM((CH, DIM), jnp.float32),
            pltpu.VMEM((CH, DIM), jnp.float32),
            pltpu.VMEM_SHARED((NP, DIM), jnp.float32),
            pltpu.SemaphoreType.DMA,
            pltpu.SemaphoreType.DMA,
        ],
        compiler_params=pltpu.CompilerParams(use_tc_tiling_on_sc=False),
    )(y, src, dst, ew, zeros)

    out = pl.pallas_call(
        _final_body,
        out_shape=jax.ShapeDtypeStruct((N_GRAPHS, 2), jnp.float32),
    )(agg_parts, x, W_root, b_rel.reshape(1, DIM), batch32,
      W1, b1.reshape(1, DIM), W2, b2.reshape(1, 2))
    return out


# speedup vs baseline: 12.6930x; 1.2374x over previous
import functools

import jax
import jax.numpy as jnp
from jax import lax
from jax.experimental import pallas as pl
from jax.experimental.pallas import tpu as pltpu
from jax.experimental.pallas import tpu_sc as plsc

N_NODES = 10000
N_EDGES = 320000
D_FEAT = 128
DIM = 32
N_GRAPHS = 64

NC = 2
NS = 16
NW = NC * NS
CH = 128
NCH = 80
EPAD = NW * NCH * CH
NP = 10240
RPS = NP // NS


def _mm_body(x_ref, wr_ref, y_ref):
    dn = (((1,), (1,)), ((), ()))
    y = lax.dot_general(x_ref[...], wr_ref[...], dn,
                        preferred_element_type=jnp.float32)
    yb = y.astype(jnp.bfloat16)
    y_ref[0] = yb
    y_ref[1] = yb


def _sc_agg_body(y_hbm, src_hbm, dst_hbm, w_hbm, zeros_hbm, out_hbm,
                 src_v, dst_v, w_v, rows0_v, rows1_v, f0_v, f1_v,
                 agg_sh, sem0, sem1):
    c = lax.axis_index("c")
    s = lax.axis_index("s")
    wid = c * NS + s
    y_c = y_hbm.at[c]

    pltpu.sync_copy(zeros_hbm.at[c, pl.ds(s * RPS, RPS)],
                    agg_sh.at[pl.ds(s * RPS, RPS)])
    plsc.subcore_barrier()

    pltpu.sync_copy(src_hbm.at[wid], src_v)
    pltpu.sync_copy(dst_hbm.at[wid], dst_v)
    pltpu.sync_copy(w_hbm.at[wid], w_v)

    mask_hi = jnp.uint32(0xFFFF0000)

    def scale(rows_v, f_v, j):
        def group_body(g, _):
            base = g * 16
            wv = w_v[j, pl.ds(base, 16)]
            for k in range(16):
                ws = wv[k]
                u = plsc.bitcast(rows_v[base + k, :], jnp.uint32)
                lo = plsc.bitcast(u << 16, jnp.float32)
                hi = plsc.bitcast(u & mask_hi, jnp.float32)
                f_v[base + k, pl.ds(0, 16)] = lo * ws
                f_v[base + k, pl.ds(16, 16)] = hi * ws
            return 0

        lax.fori_loop(0, CH // 16, group_body, 0)

    def wait(rows_v, sem):
        pltpu.make_async_copy(y_hbm.at[c, pl.ds(0, CH)], rows_v, sem).wait()

    pltpu.async_copy(y_c.at[src_v.at[0]], rows0_v, sem0)

    def pair_body(p, _):
        j0 = 2 * p
        j1 = 2 * p + 1
        pltpu.async_copy(y_c.at[src_v.at[j1]], rows1_v, sem1)
        wait(rows0_v, sem0)
        scale(rows0_v, f0_v, j0)
        pltpu.sync_copy(f0_v, agg_sh.at[dst_v.at[j0]], add=True)
        jn = jnp.minimum(j0 + 2, NCH - 1)
        pltpu.async_copy(y_c.at[src_v.at[jn]], rows0_v, sem0)
        wait(rows1_v, sem1)
        scale(rows1_v, f1_v, j1)
        pltpu.sync_copy(f1_v, agg_sh.at[dst_v.at[j1]], add=True)
        return 0

    lax.fori_loop(0, NCH // 2, pair_body, 0)
    wait(rows0_v, sem0)
    plsc.subcore_barrier()

    pltpu.sync_copy(agg_sh.at[pl.ds(s * RPS, RPS)],
                    out_hbm.at[c, pl.ds(s * RPS, RPS)])


def _final_body(agg_ref, x_ref, wo_ref, brel_ref, batch_ref,
                w1_ref, b1_ref, w2_ref, b2_ref, out_ref):
    dn = (((1,), (1,)), ((), ()))
    agg = agg_ref[0, :N_NODES, :] + agg_ref[1, :N_NODES, :]
    r = lax.dot_general(x_ref[...], wo_ref[...], dn,
                        preferred_element_type=jnp.float32)
    h = jnp.maximum(agg + brel_ref[...] + r, 0.0)

    bt = batch_ref[...]
    gids = lax.broadcasted_iota(jnp.int32, (N_GRAPHS, N_NODES), 0)
    seg = (bt == gids).astype(jnp.float32)
    pooled = lax.dot_general(seg, h, (((1,), (0,)), ((), ())),
                             preferred_element_type=jnp.float32)

    z = jnp.maximum(lax.dot_general(pooled, w1_ref[...], dn,
                                    preferred_element_type=jnp.float32)
                    + b1_ref[...], 0.0)
    o = lax.dot_general(z, w2_ref[...], dn,
                        preferred_element_type=jnp.float32) + b2_ref[...]
    m = jnp.max(o, axis=1, keepdims=True)
    lo = o - m
    out_ref[...] = lo - jnp.log(jnp.sum(jnp.exp(lo), axis=1, keepdims=True))


def kernel(x, edge_index, batch, edge_weight, W_rel, b_rel, W_root,
           W1, b1, W2, b2):
    pad = EPAD - N_EDGES
    ipad = jnp.zeros((pad,), jnp.int32)
    src = jnp.concatenate([edge_index[0].astype(jnp.int32), ipad])
    dst = jnp.concatenate([edge_index[1].astype(jnp.int32), ipad])
    ew = jnp.concatenate([edge_weight.astype(jnp.float32),
                          jnp.zeros((pad,), jnp.float32)])
    src = src.reshape(NW, NCH, CH)
    dst = dst.reshape(NW, NCH, CH)
    ew = ew.reshape(NW, NCH, CH)
    batch32 = batch.astype(jnp.int32).reshape(1, N_NODES)
    zeros = jnp.zeros((NC, NP, DIM), jnp.float32)

    y = pl.pallas_call(
        _mm_body,
        out_shape=jax.ShapeDtypeStruct((NC, N_NODES, DIM), jnp.bfloat16),
    )(x, W_rel)

    perm = jnp.array(list(range(0, DIM, 2)) + list(range(1, DIM, 2)),
                     dtype=jnp.int32)
    W_root_p = W_root[perm, :]
    b_rel_p = b_rel[perm]
    W1_p = W1[:, perm]

    mesh = plsc.VectorSubcoreMesh(core_axis_name="c", subcore_axis_name="s",
                                  num_cores=NC, num_subcores=NS)
    agg_parts = pl.kernel(
        _sc_agg_body,
        out_type=jax.ShapeDtypeStruct((NC, NP, DIM), jnp.float32),
        mesh=mesh,
        scratch_types=[
            pltpu.VMEM((NCH, CH), jnp.int32),
            pltpu.VMEM((NCH, CH), jnp.int32),
            pltpu.VMEM((NCH, CH), jnp.float32),
            pltpu.VMEM((CH, DIM), jnp.bfloat16),
            pltpu.VMEM((CH, DIM), jnp.bfloat16),
            pltpu.VMEM((CH, DIM), jnp.float32),
            pltpu.VMEM((CH, DIM), jnp.float32),
            pltpu.VMEM_SHARED((NP, DIM), jnp.float32),
            pltpu.SemaphoreType.DMA,
            pltpu.SemaphoreType.DMA,
        ],
        compiler_params=pltpu.CompilerParams(use_tc_tiling_on_sc=False,
                                             needs_layout_passes=False),
    )(y, src, dst, ew, zeros)

    out = pl.pallas_call(
        _final_body,
        out_shape=jax.ShapeDtypeStruct((N_GRAPHS, 2), jnp.float32),
    )(agg_parts, x, W_root_p, b_rel_p.reshape(1, DIM), batch32,
      W1_p, b1.reshape(1, DIM), W2, b2.reshape(1, 2))
    return out


# speedup vs baseline: 14.6307x; 1.1527x over previous
import functools

import jax
import jax.numpy as jnp
from jax import lax
from jax.experimental import pallas as pl
from jax.experimental.pallas import tpu as pltpu
from jax.experimental.pallas import tpu_sc as plsc

N_NODES = 10000
N_EDGES = 320000
D_FEAT = 128
DIM = 32
N_GRAPHS = 64

NC = 2
NS = 16
NW = NC * NS
CH = 128
NCH = 80
EPAD = NW * NCH * CH
NP = 10240
RPS = NP // NS
PROW = N_NODES * DIM // 128


def _mm_body(x_ref, wr_ref, wrt_ref, brel_ref, y_ref, r_ref):
    dn = (((1,), (1,)), ((), ()))
    x = x_ref[...]
    y = lax.dot_general(x, wr_ref[...], dn,
                        preferred_element_type=jnp.float32)
    y_ref[...] = y.astype(jnp.bfloat16)
    r = lax.dot_general(x, wrt_ref[...], dn,
                        preferred_element_type=jnp.float32) + brel_ref[...]
    r_ref[...] = jnp.concatenate(
        [r, jnp.zeros((NP - N_NODES, DIM), jnp.float32)], axis=0)


def _sc_agg_body(y_hbm, ei_hbm, w_hbm, r_hbm, out_hbm,
                 src_v, dst_v, w_v, rows0_v, rows1_v, f0_v, f1_v,
                 agg_sh, sem0, sem1):
    c = lax.axis_index("c")
    s = lax.axis_index("s")
    wid = c * NS + s

    @pl.when(c == 0)
    def _():
        pltpu.sync_copy(r_hbm.at[pl.ds(s * RPS, RPS)],
                        agg_sh.at[pl.ds(s * RPS, RPS)])

    @pl.when(c != 0)
    def _():
        zv = jnp.zeros((16,), jnp.float32)

        def zrow(i, _):
            f0_v[i, pl.ds(0, 16)] = zv
            f0_v[i, pl.ds(16, 16)] = zv
            return 0

        lax.fori_loop(0, CH, zrow, 0)
        for q in range(RPS // CH):
            pltpu.sync_copy(f0_v, agg_sh.at[pl.ds(s * RPS + q * CH, CH)])

    plsc.subcore_barrier()

    pltpu.sync_copy(ei_hbm.at[wid], src_v)
    pltpu.sync_copy(ei_hbm.at[NW + wid], dst_v)
    pltpu.sync_copy(w_hbm.at[wid], w_v)

    mask_hi = jnp.uint32(0xFFFF0000)

    def scale(rows_v, f_v, j):
        def group_body(g, _):
            base = g * 16
            wv = w_v[j, pl.ds(base, 16)]
            for k in range(16):
                ws = wv[k]
                u = plsc.bitcast(rows_v[base + k, :], jnp.uint32)
                lo = plsc.bitcast(u << 16, jnp.float32)
                hi = plsc.bitcast(u & mask_hi, jnp.float32)
                f_v[base + k, pl.ds(0, 16)] = lo * ws
                f_v[base + k, pl.ds(16, 16)] = hi * ws
            return 0

        lax.fori_loop(0, CH // 16, group_body, 0)

    def wait(rows_v, sem):
        pltpu.make_async_copy(y_hbm.at[pl.ds(0, CH)], rows_v, sem).wait()

    pltpu.async_copy(y_hbm.at[src_v.at[0]], rows0_v, sem0)

    def pair_body(p, _):
        j0 = 2 * p
        j1 = 2 * p + 1
        pltpu.async_copy(y_hbm.at[src_v.at[j1]], rows1_v, sem1)
        wait(rows0_v, sem0)
        scale(rows0_v, f0_v, j0)
        pltpu.sync_copy(f0_v, agg_sh.at[dst_v.at[j0]], add=True)
        jn = jnp.minimum(j0 + 2, NCH - 1)
        pltpu.async_copy(y_hbm.at[src_v.at[jn]], rows0_v, sem0)
        wait(rows1_v, sem1)
        scale(rows1_v, f1_v, j1)
        pltpu.sync_copy(f1_v, agg_sh.at[dst_v.at[j1]], add=True)
        return 0

    lax.fori_loop(0, NCH // 2, pair_body, 0)
    wait(rows0_v, sem0)
    plsc.subcore_barrier()

    pltpu.sync_copy(agg_sh.at[pl.ds(s * RPS, RPS)],
                    out_hbm.at[c, pl.ds(s * RPS, RPS)])


def _final_body(agg_ref, batch_ref, w1_ref, b1_ref, w2_ref, b2_ref,
                out_ref):
    dn = (((1,), (1,)), ((), ()))
    h = jnp.maximum(agg_ref[0] + agg_ref[1], 0.0)[:PROW, :]
    gids = lax.broadcasted_iota(jnp.int32, (N_GRAPHS, PROW), 0)
    pooled = jnp.zeros((N_GRAPHS, DIM), jnp.float32)
    for m in range(4):
        seg = (batch_ref[m][None, :] == gids).astype(jnp.float32)
        pm = lax.dot_general(seg, h, (((1,), (0,)), ((), ())),
                             preferred_element_type=jnp.float32)
        pooled = pooled + pm[:, 32 * m:32 * (m + 1)]

    z = jnp.maximum(lax.dot_general(pooled, w1_ref[...], dn,
                                    preferred_element_type=jnp.float32)
                    + b1_ref[...], 0.0)
    o = lax.dot_general(z, w2_ref[...], dn,
                        preferred_element_type=jnp.float32) + b2_ref[...]
    m = jnp.max(o, axis=1, keepdims=True)
    lo = o - m
    out_ref[...] = lo - jnp.log(jnp.sum(jnp.exp(lo), axis=1, keepdims=True))


def kernel(x, edge_index, batch, edge_weight, W_rel, b_rel, W_root,
           W1, b1, W2, b2):
    pad = EPAD - N_EDGES
    ei = jnp.pad(edge_index.astype(jnp.int32), ((0, 0), (0, pad)))
    ei = ei.reshape(2 * NW, NCH, CH)
    ew = jnp.pad(edge_weight.astype(jnp.float32), (0, pad))
    ew = ew.reshape(NW, NCH, CH)
    batch4 = batch.astype(jnp.int32).reshape(PROW, 4).T

    perm = jnp.array(list(range(0, DIM, 2)) + list(range(1, DIM, 2)),
                     dtype=jnp.int32)
    W_root_p = W_root[perm, :]
    b_rel_p = b_rel[perm]
    W1_p = W1[:, perm]

    y, r_init = pl.pallas_call(
        _mm_body,
        out_shape=[jax.ShapeDtypeStruct((N_NODES, DIM), jnp.bfloat16),
                   jax.ShapeDtypeStruct((NP, DIM), jnp.float32)],
    )(x, W_rel, W_root_p, b_rel_p.reshape(1, DIM))

    mesh = plsc.VectorSubcoreMesh(core_axis_name="c", subcore_axis_name="s",
                                  num_cores=NC, num_subcores=NS)
    agg_parts = pl.kernel(
        _sc_agg_body,
        out_type=jax.ShapeDtypeStruct((NC, NP, DIM), jnp.float32),
        mesh=mesh,
        scratch_types=[
            pltpu.VMEM((NCH, CH), jnp.int32),
            pltpu.VMEM((NCH, CH), jnp.int32),
            pltpu.VMEM((NCH, CH), jnp.float32),
            pltpu.VMEM((CH, DIM), jnp.bfloat16),
            pltpu.VMEM((CH, DIM), jnp.bfloat16),
            pltpu.VMEM((CH, DIM), jnp.float32),
            pltpu.VMEM((CH, DIM), jnp.float32),
            pltpu.VMEM_SHARED((NP, DIM), jnp.float32),
            pltpu.SemaphoreType.DMA,
            pltpu.SemaphoreType.DMA,
        ],
        compiler_params=pltpu.CompilerParams(use_tc_tiling_on_sc=False,
                                             needs_layout_passes=False),
    )(y, ei, ew, r_init)

    aggp = agg_parts.reshape(NC, NP * DIM // 128, 128)

    out = pl.pallas_call(
        _final_body,
        out_shape=jax.ShapeDtypeStruct((N_GRAPHS, 2), jnp.float32),
    )(aggp, batch4, W1_p, b1.reshape(1, DIM), W2, b2.reshape(1, 2))
    return out


# speedup vs baseline: 15.4998x; 1.0594x over previous
import functools

import jax
import jax.numpy as jnp
from jax import lax
from jax.experimental import pallas as pl
from jax.experimental.pallas import tpu as pltpu
from jax.experimental.pallas import tpu_sc as plsc

N_NODES = 10000
N_EDGES = 320000
D_FEAT = 128
DIM = 32
N_GRAPHS = 64

NC = 2
NS = 16
NW = NC * NS
CH = 128
NCH = 80
EPAD = NW * NCH * CH
NP = 10240
RPS = NP // NS
PROW = N_NODES * DIM // 128


def _mm_body(x_ref, wr_ref, wrt_ref, brel_ref, y_ref, r_ref):
    dn = (((1,), (1,)), ((), ()))
    x = x_ref[...]
    y = lax.dot_general(x, wr_ref[...], dn,
                        preferred_element_type=jnp.float32)
    y_ref[...] = y.astype(jnp.bfloat16)
    r = lax.dot_general(x, wrt_ref[...], dn,
                        preferred_element_type=jnp.float32) + brel_ref[...]
    r_ref[...] = jnp.concatenate(
        [r, jnp.zeros((NP - N_NODES, DIM), jnp.float32)], axis=0)


def _sc_agg_body(y_hbm, ei_hbm, w_hbm, r_hbm, out_hbm,
                 src_v, dst_v, w_v, rows0_v, rows1_v, f0_v, f1_v,
                 agg_sh, sem0, sem1, sems0, sems1):
    c = lax.axis_index("c")
    s = lax.axis_index("s")
    wid = c * NS + s

    @pl.when(c == 0)
    def _():
        pltpu.sync_copy(r_hbm.at[pl.ds(s * RPS, RPS)],
                        agg_sh.at[pl.ds(s * RPS, RPS)])

    @pl.when(c != 0)
    def _():
        zv = jnp.zeros((16,), jnp.float32)

        def zrow(i, _):
            f0_v[i, pl.ds(0, 16)] = zv
            f0_v[i, pl.ds(16, 16)] = zv
            return 0

        lax.fori_loop(0, CH, zrow, 0)
        for q in range(RPS // CH):
            pltpu.sync_copy(f0_v, agg_sh.at[pl.ds(s * RPS + q * CH, CH)])

    plsc.subcore_barrier()

    pltpu.sync_copy(ei_hbm.at[wid], src_v)
    pltpu.sync_copy(ei_hbm.at[NW + wid], dst_v)
    pltpu.sync_copy(w_hbm.at[wid], w_v)

    mask_hi = jnp.uint32(0xFFFF0000)

    def scale(rows_v, f_v, j):
        def group_body(g, _):
            base = g * 16
            wv = w_v[j, pl.ds(base, 16)]
            for k in range(16):
                ws = wv[k]
                u = plsc.bitcast(rows_v[base + k, :], jnp.uint32)
                lo = plsc.bitcast(u << 16, jnp.float32)
                hi = plsc.bitcast(u & mask_hi, jnp.float32)
                f_v[base + k, pl.ds(0, 16)] = lo * ws
                f_v[base + k, pl.ds(16, 16)] = hi * ws
            return 0

        lax.fori_loop(0, CH // 16, group_body, 0)

    def wait(rows_v, sem):
        pltpu.make_async_copy(y_hbm.at[pl.ds(0, CH)], rows_v, sem).wait()

    pltpu.async_copy(y_hbm.at[src_v.at[0]], rows0_v, sem0)

    def wait_sc(f_v, sem):
        pltpu.make_async_copy(f_v, agg_sh.at[pl.ds(0, CH)], sem).wait()

    def pair_body(p, _):
        j0 = 2 * p
        j1 = 2 * p + 1
        pltpu.async_copy(y_hbm.at[src_v.at[j1]], rows1_v, sem1)
        wait(rows0_v, sem0)

        @pl.when(p > 0)
        def _():
            wait_sc(f0_v, sems0)

        scale(rows0_v, f0_v, j0)
        pltpu.async_copy(f0_v, agg_sh.at[dst_v.at[j0]], sems0, add=True)
        jn = jnp.minimum(j0 + 2, NCH - 1)
        pltpu.async_copy(y_hbm.at[src_v.at[jn]], rows0_v, sem0)
        wait(rows1_v, sem1)

        @pl.when(p > 0)
        def _():
            wait_sc(f1_v, sems1)

        scale(rows1_v, f1_v, j1)
        pltpu.async_copy(f1_v, agg_sh.at[dst_v.at[j1]], sems1, add=True)
        return 0

    lax.fori_loop(0, NCH // 2, pair_body, 0)
    wait(rows0_v, sem0)
    wait_sc(f0_v, sems0)
    wait_sc(f1_v, sems1)
    plsc.subcore_barrier()

    pltpu.sync_copy(agg_sh.at[pl.ds(s * RPS, RPS)],
                    out_hbm.at[c, pl.ds(s * RPS, RPS)])


def _final_body(agg_ref, batch_ref, w1_ref, b1_ref, w2_ref, b2_ref,
                out_ref):
    dn = (((1,), (1,)), ((), ()))
    h = jnp.maximum(agg_ref[0] + agg_ref[1], 0.0)[:PROW, :]
    gids = lax.broadcasted_iota(jnp.int32, (N_GRAPHS, PROW), 0)
    pooled = jnp.zeros((N_GRAPHS, DIM), jnp.float32)
    for m in range(4):
        seg = (batch_ref[m][None, :] == gids).astype(jnp.float32)
        pm = lax.dot_general(seg, h, (((1,), (0,)), ((), ())),
                             preferred_element_type=jnp.float32)
        pooled = pooled + pm[:, 32 * m:32 * (m + 1)]

    z = jnp.maximum(lax.dot_general(pooled, w1_ref[...], dn,
                                    preferred_element_type=jnp.float32)
                    + b1_ref[...], 0.0)
    o = lax.dot_general(z, w2_ref[...], dn,
                        preferred_element_type=jnp.float32) + b2_ref[...]
    m = jnp.max(o, axis=1, keepdims=True)
    lo = o - m
    out_ref[...] = lo - jnp.log(jnp.sum(jnp.exp(lo), axis=1, keepdims=True))


def kernel(x, edge_index, batch, edge_weight, W_rel, b_rel, W_root,
           W1, b1, W2, b2):
    pad = EPAD - N_EDGES
    ei = jnp.pad(edge_index.astype(jnp.int32), ((0, 0), (0, pad)))
    ei = ei.reshape(2 * NW, NCH, CH)
    ew = jnp.pad(edge_weight.astype(jnp.float32), (0, pad))
    ew = ew.reshape(NW, NCH, CH)
    batch4 = batch.astype(jnp.int32).reshape(PROW, 4).T

    perm = jnp.array(list(range(0, DIM, 2)) + list(range(1, DIM, 2)),
                     dtype=jnp.int32)
    W_root_p = W_root[perm, :]
    b_rel_p = b_rel[perm]
    W1_p = W1[:, perm]

    y, r_init = pl.pallas_call(
        _mm_body,
        out_shape=[jax.ShapeDtypeStruct((N_NODES, DIM), jnp.bfloat16),
                   jax.ShapeDtypeStruct((NP, DIM), jnp.float32)],
    )(x, W_rel, W_root_p, b_rel_p.reshape(1, DIM))

    mesh = plsc.VectorSubcoreMesh(core_axis_name="c", subcore_axis_name="s",
                                  num_cores=NC, num_subcores=NS)
    agg_parts = pl.kernel(
        _sc_agg_body,
        out_type=jax.ShapeDtypeStruct((NC, NP, DIM), jnp.float32),
        mesh=mesh,
        scratch_types=[
            pltpu.VMEM((NCH, CH), jnp.int32),
            pltpu.VMEM((NCH, CH), jnp.int32),
            pltpu.VMEM((NCH, CH), jnp.float32),
            pltpu.VMEM((CH, DIM), jnp.bfloat16),
            pltpu.VMEM((CH, DIM), jnp.bfloat16),
            pltpu.VMEM((CH, DIM), jnp.float32),
            pltpu.VMEM((CH, DIM), jnp.float32),
            pltpu.VMEM_SHARED((NP, DIM), jnp.float32),
            pltpu.SemaphoreType.DMA,
            pltpu.SemaphoreType.DMA,
            pltpu.SemaphoreType.DMA,
            pltpu.SemaphoreType.DMA,
        ],
        compiler_params=pltpu.CompilerParams(use_tc_tiling_on_sc=False,
                                             needs_layout_passes=False),
    )(y, ei, ew, r_init)

    aggp = agg_parts.reshape(NC, NP * DIM // 128, 128)

    out = pl.pallas_call(
        _final_body,
        out_shape=jax.ShapeDtypeStruct((N_GRAPHS, 2), jnp.float32),
    )(aggp, batch4, W1_p, b1.reshape(1, DIM), W2, b2.reshape(1, 2))
    return out
